# baseline (speedup 1.0000x reference)
import functools

import jax
import jax.numpy as jnp
from jax import lax
from jax.experimental import pallas as pl
from jax.experimental.pallas import tpu as pltpu
from jax.experimental.pallas import tpu_sc as plsc

_B = 4096
_P = 361
_D = 128

_NC = 2
_NS = 16
_NP = 12
_NB = 8
_NSLOT = 6
_NBLK = _B // _NB
_NRING = _NBLK // _NSLOT
_NTAIL = _NBLK - _NRING * _NSLOT
_RPB = _NB * _NP
_RG = _RPB // 16
_W = _NP * _D


def _sc_kernel(boards_f, tab_f, out_hbm, tloc, bbuf, obuf, wsem, bsem):
    wid = lax.axis_index("s") * _NC + lax.axis_index("c")
    p0 = jnp.minimum(wid * _NP, _P - _NP)
    lane = lax.iota(jnp.int32, 16)

    pltpu.sync_copy(tab_f.at[pl.ds(p0 * 3 * _D, _NP * 3 * _D)], tloc)
    for k in range(_NP):
        pltpu.async_copy(boards_f.at[pl.ds((p0 + k) * _B, _B)],
                         bbuf.at[pl.ds(k * _B, _B)], bsem)
    for k in range(_NP):
        pltpu.make_async_copy(boards_f.at[pl.ds(0, _B)],
                              bbuf.at[pl.ds(0, _B)], bsem).wait()

    blocs, prels, wofs, tofs = [], [], [], []
    for rg in range(_RG):
        rvec = rg * 16 + lane
        prel = lax.rem(rvec, _NP)
        blocs.append(lax.div(rvec, jnp.int32(_NP)))
        prels.append(prel)
        wofs.append(prel * _D)
        tofs.append(prel * (3 * _D))

    def do_block(blk, slot, reclaim):
        babs0 = blk * _NB

        @pl.when(reclaim)
        def _():
            pltpu.make_async_copy(
                obuf.at[pl.ds(slot * _NB, _NB)],
                out_hbm.at[pl.ds(0, _NB), pl.ds(0, _W)],
                wsem.at[slot]).wait()

        def rbody(r, carry):
            bloc = lax.div(r, jnp.int32(_NP))
            prel = lax.rem(r, jnp.int32(_NP))
            bval = bbuf[pl.ds(prel * _B + babs0 + bloc, 16)][0]
            rowstart = (prel * 3 + bval) * _D
            srow = slot * _NB + bloc
            col0 = prel * _D
            for j in range(_D // 16):
                obuf[srow, pl.ds(col0 + j * 16, 16)] = (
                    tloc[pl.ds(rowstart + j * 16, 16)])
            return carry

        lax.fori_loop(0, _RPB, rbody, 0)

        pltpu.async_copy(
            obuf.at[pl.ds(slot * _NB, _NB)],
            out_hbm.at[pl.ds(babs0, _NB), pl.ds(p0 * _D, _W)],
            wsem.at[slot])

    def body(s, carry):
        for slot in range(_NSLOT):
            do_block(s * _NSLOT + slot, slot, s > 0)
        return carry

    lax.fori_loop(0, _NRING, body, 0)
    for t in range(_NTAIL):
        do_block(_NRING * _NSLOT + t, t, jnp.bool_(True))

    for slot in range(_NSLOT):
        pltpu.make_async_copy(
            obuf.at[pl.ds(slot * _NB, _NB)],
            out_hbm.at[pl.ds(0, _NB), pl.ds(0, _W)],
            wsem.at[slot]).wait()


@jax.jit
def _lookup(boards_f, tab_f):
    mesh = plsc.VectorSubcoreMesh(core_axis_name="c", subcore_axis_name="s")
    f = functools.partial(
        pl.kernel,
        mesh=mesh,
        out_type=jax.ShapeDtypeStruct((_B, _P * _D), jnp.float32),
        scratch_types=[
            pltpu.VMEM((_NP * 3 * _D,), jnp.float32),
            pltpu.VMEM((_NP * _B + 16,), jnp.int32),
            pltpu.VMEM((_NSLOT * _NB, _W), jnp.float32),
            pltpu.SemaphoreType.DMA((_NSLOT,)),
            pltpu.SemaphoreType.DMA,
        ],
    )(_sc_kernel)
    return f(boards_f, tab_f)


def kernel(boards, embeddings):
    boards_f = boards.T.reshape(_P * _B)
    tab_f = embeddings.reshape(_P * 3 * _D)
    out = _lookup(boards_f, tab_f)
    return out.reshape(_B, _P, _D)


# speedup vs baseline: 1.1718x; 1.1718x over previous
import functools

import jax
import jax.numpy as jnp
from jax import lax
from jax.experimental import pallas as pl
from jax.experimental.pallas import tpu as pltpu
from jax.experimental.pallas import tpu_sc as plsc

_B = 4096
_P = 361
_D = 128

_NC = 2
_NS = 16
_NP = 12
_NB = 8
_NSLOT = 6
_NBLK = _B // _NB
_NRING = _NBLK // _NSLOT
_NTAIL = _NBLK - _NRING * _NSLOT
_RPB = _NB * _NP
_RG = _RPB // 16
_W = _NP * _D


def _sc_kernel(boards_f, tab_f, out_hbm, tloc, bbuf, obuf, wsem, bsem):
    wid = lax.axis_index("s") * _NC + lax.axis_index("c")
    p0 = jnp.minimum(wid * _NP, _P - _NP)
    lane = lax.iota(jnp.int32, 16)

    pltpu.sync_copy(tab_f.at[pl.ds(p0 * 3 * _D, _NP * 3 * _D)], tloc)
    for k in range(_NP):
        pltpu.async_copy(boards_f.at[pl.ds((p0 + k) * _B, _B)],
                         bbuf.at[pl.ds(k * _B, _B)], bsem)
    for k in range(_NP):
        pltpu.make_async_copy(boards_f.at[pl.ds(0, _B)],
                              bbuf.at[pl.ds(0, _B)], bsem).wait()

    blocs, prels, wofs, tofs = [], [], [], []
    for rg in range(_RG):
        rvec = rg * 16 + lane
        prel = lax.rem(rvec, _NP)
        blocs.append(lax.div(rvec, jnp.int32(_NP)))
        prels.append(prel)
        wofs.append(prel * _D)
        tofs.append(prel * (3 * _D))

    def do_block(blk, slot, reclaim):
        babs0 = blk * _NB

        @pl.when(reclaim)
        def _():
            pltpu.make_async_copy(
                obuf.at[pl.ds(slot * _NB, _NB)],
                out_hbm.at[pl.ds(0, _NB), pl.ds(0, _W)],
                wsem.at[slot]).wait()

        def rbody(g, carry):
            rows = []
            for dr in range(4):
                r = g * 4 + dr
                bloc = lax.div(r, jnp.int32(_NP))
                prel = lax.rem(r, jnp.int32(_NP))
                bval = bbuf[pl.ds(prel * _B + babs0 + bloc, 16)][0]
                rows.append(((prel * 3 + bval) * _D,
                             slot * _NB + bloc, prel * _D))
            for rowstart, srow, col0 in rows:
                for j in range(_D // 16):
                    obuf[srow, pl.ds(col0 + j * 16, 16)] = (
                        tloc[pl.ds(rowstart + j * 16, 16)])
            return carry

        lax.fori_loop(0, _RPB // 4, rbody, 0)

        pltpu.async_copy(
            obuf.at[pl.ds(slot * _NB, _NB)],
            out_hbm.at[pl.ds(babs0, _NB), pl.ds(p0 * _D, _W)],
            wsem.at[slot])

    def body(s, carry):
        for slot in range(_NSLOT):
            do_block(s * _NSLOT + slot, slot, s > 0)
        return carry

    lax.fori_loop(0, _NRING, body, 0)
    for t in range(_NTAIL):
        do_block(_NRING * _NSLOT + t, t, jnp.bool_(True))

    for slot in range(_NSLOT):
        pltpu.make_async_copy(
            obuf.at[pl.ds(slot * _NB, _NB)],
            out_hbm.at[pl.ds(0, _NB), pl.ds(0, _W)],
            wsem.at[slot]).wait()


@jax.jit
def _lookup(boards_f, tab_f):
    mesh = plsc.VectorSubcoreMesh(core_axis_name="c", subcore_axis_name="s")
    f = functools.partial(
        pl.kernel,
        mesh=mesh,
        out_type=jax.ShapeDtypeStruct((_B, _P * _D), jnp.float32),
        scratch_types=[
            pltpu.VMEM((_NP * 3 * _D,), jnp.float32),
            pltpu.VMEM((_NP * _B + 16,), jnp.int32),
            pltpu.VMEM((_NSLOT * _NB, _W), jnp.float32),
            pltpu.SemaphoreType.DMA((_NSLOT,)),
            pltpu.SemaphoreType.DMA,
        ],
    )(_sc_kernel)
    return f(boards_f, tab_f)


def kernel(boards, embeddings):
    boards_f = boards.T.reshape(_P * _B)
    tab_f = embeddings.reshape(_P * 3 * _D)
    out = _lookup(boards_f, tab_f)
    return out.reshape(_B, _P, _D)


# speedup vs baseline: 2.1696x; 1.8514x over previous
import functools

import jax
import jax.numpy as jnp
from jax import lax
from jax.experimental import pallas as pl
from jax.experimental.pallas import tpu as pltpu
from jax.experimental.pallas import tpu_sc as plsc

_B = 4096
_P = 361
_D = 128
_N = _B * _P

_NC = 2
_NS = 16
_NW = _NC * _NS
_NL = _N // _NW
_CH = 128
_GPW = _NL // _CH
_K = 3
_NBLK = (_GPW // (2 * _K)) * 2
_BCH = _K * _CH


def _sc_gather_kernel(boards_hbm, tab_hbm, out_hbm,
                      tab_sp, bblk_v, idx_a, idx_b, rows_a, rows_b,
                      gsem_a, gsem_b, wsem_a, wsem_b):
    sid = lax.axis_index("s")
    wid = sid * _NC + lax.axis_index("c")
    base = wid * _NL
    lane = lax.iota(jnp.int32, 16)

    @pl.when(sid == 0)
    def _():
        pltpu.sync_copy(tab_hbm, tab_sp)
    plsc.subcore_barrier()

    def compute_idx(idx_ref, c, g):
        for j in range(_CH // 16):
            l = g * _CH + j * 16 + lane
            pos = lax.rem(l, _P)
            idx_ref[c, pl.ds(j * 16, 16)] = (
                pos * 3 + bblk_v[pl.ds(c * _CH + j * 16, 16)])

    def do_block(blk_id, idx_ref, rows_ref, gsem, wsem, s):
        off0 = base + blk_id * _BCH

        @pl.when(s > 0)
        def _():
            pltpu.make_async_copy(
                rows_ref, out_hbm.at[pl.ds(0, _BCH)], wsem).wait()

        pltpu.sync_copy(boards_hbm.at[pl.ds(off0, _BCH)], bblk_v)
        handles = []
        for c in range(_K):
            compute_idx(idx_ref, c, blk_id * _K + c)
            handles.append(pltpu.async_copy(
                tab_sp.at[idx_ref.at[c]],
                rows_ref.at[pl.ds(c * _CH, _CH)], gsem))
        for h in handles:
            h.wait()
        pltpu.async_copy(rows_ref, out_hbm.at[pl.ds(off0, _BCH)], wsem)

    def body(s, carry):
        do_block(2 * s, idx_a, rows_a, gsem_a, wsem_a, s)
        do_block(2 * s + 1, idx_b, rows_b, gsem_b, wsem_b, s)
        return carry

    lax.fori_loop(0, _NBLK // 2, body, 0)

    pltpu.make_async_copy(rows_a, out_hbm.at[pl.ds(0, _BCH)], wsem_a).wait()
    pltpu.make_async_copy(rows_b, out_hbm.at[pl.ds(0, _BCH)], wsem_b).wait()

    offt = base + _NBLK * _BCH
    pltpu.sync_copy(boards_hbm.at[pl.ds(offt, _CH)],
                    bblk_v.at[pl.ds(0, _CH)])
    compute_idx(idx_a, 0, _NBLK * _K)
    pltpu.async_copy(tab_sp.at[idx_a.at[0]],
                     rows_a.at[pl.ds(0, _CH)], gsem_a).wait()
    pltpu.sync_copy(rows_a.at[pl.ds(0, _CH)], out_hbm.at[pl.ds(offt, _CH)])


@jax.jit
def _lookup(boards_flat, table):
    mesh = plsc.VectorSubcoreMesh(core_axis_name="c", subcore_axis_name="s")
    f = functools.partial(
        pl.kernel,
        mesh=mesh,
        out_type=jax.ShapeDtypeStruct((_N, _D), jnp.float32),
        scratch_types=[
            pltpu.VMEM_SHARED((_P * 3, _D), jnp.float32),
            pltpu.VMEM((_BCH,), jnp.int32),
            pltpu.VMEM((_K, _CH), jnp.int32),
            pltpu.VMEM((_K, _CH), jnp.int32),
            pltpu.VMEM((_BCH, _D), jnp.float32),
            pltpu.VMEM((_BCH, _D), jnp.float32),
            pltpu.SemaphoreType.DMA,
            pltpu.SemaphoreType.DMA,
            pltpu.SemaphoreType.DMA,
            pltpu.SemaphoreType.DMA,
        ],
    )(_sc_gather_kernel)
    return f(boards_flat, table)


def kernel(boards, embeddings):
    boards_flat = boards.reshape(_N)
    table = embeddings.reshape(_P * 3, _D)
    out = _lookup(boards_flat, table)
    return out.reshape(_B, _P, _D)
